# baseline (speedup 1.0000x reference)
import functools

import jax
import jax.numpy as jnp
from jax import lax
from jax.experimental import pallas as pl
from jax.experimental.pallas import tpu as pltpu
from jax.experimental.pallas import tpu_sc as plsc

LANES = 16


@functools.lru_cache(maxsize=None)
def _build(B, S, V, D, TV):
    info = plsc.get_sparse_core_info()
    NC, NS = info.num_cores, info.num_subcores
    NW = NC * NS
    assert S % NW == 0
    SEQW = S // NW
    SEQCH = 8
    assert SEQW % SEQCH == 0
    NCHUNK = SEQW // SEQCH
    DCH = D // LANES

    mesh = plsc.VectorSubcoreMesh(core_axis_name="c", subcore_axis_name="s")

    bcast_dnums = lax.GatherDimensionNumbers(
        offset_dims=(), collapsed_slice_dims=(0,), start_index_map=(0,))

    @functools.partial(
        pl.kernel,
        mesh=mesh,
        out_type=jax.ShapeDtypeStruct((B, S, D), jnp.float32),
        scratch_types=[
            pltpu.VMEM((B, SEQW), jnp.int32),
            pltpu.VMEM((B, SEQW + LANES), jnp.int32),
            pltpu.VMEM((B * SEQCH, D), jnp.float32),
            pltpu.VMEM((B * SEQCH, D), jnp.float32),
            pltpu.VMEM((B * SEQCH, D), jnp.float32),
            pltpu.VMEM((SEQCH, D), jnp.float32),
            pltpu.VMEM((SEQCH, D), jnp.float32),
            pltpu.VMEM((TV, D), jnp.float32),
            pltpu.VMEM((D,), jnp.float32),
            pltpu.SemaphoreType.DMA,
            pltpu.SemaphoreType.DMA,
            pltpu.SemaphoreType.DMA,
            pltpu.SemaphoreType.DMA,
            pltpu.SemaphoreType.DMA,
            pltpu.SemaphoreType.DMA,
            pltpu.SemaphoreType.DMA,
            pltpu.SemaphoreType.DMA,
            pltpu.SemaphoreType.DMA,
        ],
    )
    def emb(ids_hbm, tt_hbm, table_hbm, seg_hbm, pe_hbm, out_hbm,
            idx2d, tt2d, g0, g1, g2, p0, p1, seg_v, dlt_v,
            sg0, sg1, sg2, sp0, sp1, so0, so1, so2, s_setup):
        gbuf = (g0, g1, g2)
        pbuf = (p0, p1)
        sg = (sg0, sg1, sg2)
        sp = (sp0, sp1)
        so = (so0, so1, so2)

        wid = lax.axis_index("s") * NC + lax.axis_index("c")
        sq0 = wid * SEQW

        setup = []
        for b in range(B):
            setup.append(pltpu.make_async_copy(
                ids_hbm.at[b, pl.ds(sq0, SEQW)], idx2d.at[b], s_setup))
            setup.append(pltpu.make_async_copy(
                tt_hbm.at[b, pl.ds(sq0, SEQW)],
                tt2d.at[b, pl.ds(0, SEQW)], s_setup))
        setup.append(pltpu.make_async_copy(seg_hbm, seg_v, s_setup))
        for cp in setup:
            cp.start()

        def p_copy(c):
            return pltpu.make_async_copy(
                pe_hbm.at[pl.ds(sq0 + c * SEQCH, SEQCH)], pbuf[c % 2],
                sp[c % 2])

        p_copy(0).start()
        p_copy(1).start()

        for cp in setup:
            cp.wait()

        def g_copies(c):
            r = c % 3
            return [
                pltpu.make_async_copy(
                    table_hbm.at[idx2d.at[b, pl.ds(c * SEQCH, SEQCH)]],
                    gbuf[r].at[pl.ds(b * SEQCH, SEQCH)], sg[r])
                for b in range(B)
            ]

        def o_copies(c):
            r = c % 3
            return [
                pltpu.make_async_copy(
                    gbuf[r].at[pl.ds(b * SEQCH, SEQCH)],
                    out_hbm.at[b, pl.ds(sq0 + c * SEQCH, SEQCH)], so[r])
                for b in range(B)
            ]

        for cp in g_copies(0):
            cp.start()

        def dlt(j, _):
            sl = pl.ds(j * LANES, LANES)
            dlt_v[sl] = seg_v[1, sl] - seg_v[0, sl]
            return 0

        lax.fori_loop(0, DCH, dlt, 0)

        for c in range(NCHUNK):
            r = c % 3
            if c >= 2:
                for cp in o_copies(c - 2):
                    cp.wait()
            if c + 1 < NCHUNK:
                for cp in g_copies(c + 1):
                    cp.start()
            for cp in g_copies(c):
                cp.wait()
            p_copy(c).wait()

            gb = gbuf[r]
            pb = pbuf[c % 2]
            ttvs = [tt2d[b, pl.ds(c * SEQCH, LANES)].astype(jnp.float32)
                    for b in range(B)]

            def body_at(j):
                sl = pl.ds(j * LANES, LANES)
                s0v = seg_v[0, sl]
                dv = dlt_v[sl]
                pek = [pb[k, sl] + s0v for k in range(SEQCH)]
                for b in range(B):
                    for k in range(SEQCH):
                        f = lax.gather(
                            ttvs[b], jnp.full((LANES, 1), k, jnp.int32),
                            bcast_dnums, (1,),
                            mode=lax.GatherScatterMode.PROMISE_IN_BOUNDS)
                        i = b * SEQCH + k
                        gb[i, sl] = gb[i, sl] + pek[k] + f * dv

            def dloop(j, _):
                body_at(2 * j)
                body_at(2 * j + 1)
                return 0

            lax.fori_loop(0, DCH // 2, dloop, 0)

            for cp in o_copies(c):
                cp.start()
            if c + 2 < NCHUNK:
                p_copy(c + 2).start()

        for c in (NCHUNK - 2, NCHUNK - 1):
            for cp in o_copies(c):
                cp.wait()

    return emb


def kernel(input_ids, token_type_ids, token_table, segment_table, pe):
    B, S = input_ids.shape
    V, D = token_table.shape
    TV = segment_table.shape[0]
    ids = input_ids if input_ids.dtype == jnp.int32 else (
        input_ids.astype(jnp.int32))
    tt = token_type_ids if token_type_ids.dtype == jnp.int32 else (
        token_type_ids.astype(jnp.int32))
    emb = _build(B, S, V, D, TV)
    return emb(ids, tt, token_table, segment_table, pe)


# speedup vs baseline: 1.5663x; 1.5663x over previous
import functools

import jax
import jax.numpy as jnp
from jax import lax
from jax.experimental import pallas as pl
from jax.experimental.pallas import tpu as pltpu
from jax.experimental.pallas import tpu_sc as plsc

LANES = 16


@functools.lru_cache(maxsize=None)
def _build(B, S, V, D, TV):
    info = plsc.get_sparse_core_info()
    NC, NS = info.num_cores, info.num_subcores
    NW = NC * NS
    assert S % NW == 0
    SEQW = S // NW
    SEQCH = 8
    assert SEQW % SEQCH == 0
    NCHUNK = SEQW // SEQCH
    DCH = D // LANES

    mesh = plsc.VectorSubcoreMesh(core_axis_name="c", subcore_axis_name="s")

    bcast_dnums = lax.GatherDimensionNumbers(
        offset_dims=(), collapsed_slice_dims=(0,), start_index_map=(0,))

    @functools.partial(
        pl.kernel,
        mesh=mesh,
        out_type=jax.ShapeDtypeStruct((B, S, D), jnp.float32),
        scratch_types=[
            pltpu.VMEM((B, SEQW), jnp.int32),
            pltpu.VMEM((B, SEQW + LANES), jnp.int32),
            pltpu.VMEM((B * SEQCH, D), jnp.float32),
            pltpu.VMEM((B * SEQCH, D), jnp.float32),
            pltpu.VMEM((B * SEQCH, D), jnp.float32),
            pltpu.VMEM((SEQCH, D), jnp.float32),
            pltpu.VMEM((SEQCH, D), jnp.float32),
            pltpu.VMEM((TV, D), jnp.float32),
            pltpu.VMEM((D,), jnp.float32),
            pltpu.SemaphoreType.DMA,
            pltpu.SemaphoreType.DMA,
            pltpu.SemaphoreType.DMA,
            pltpu.SemaphoreType.DMA,
            pltpu.SemaphoreType.DMA,
            pltpu.SemaphoreType.DMA,
            pltpu.SemaphoreType.DMA,
            pltpu.SemaphoreType.DMA,
            pltpu.SemaphoreType.DMA,
        ],
    )
    def emb(ids_hbm, tt_hbm, table_hbm, seg_hbm, pe_hbm, out_hbm,
            idx2d, tt2d, g0, g1, g2, p0, p1, seg_v, dlt_v,
            sg0, sg1, sg2, sp0, sp1, so0, so1, so2, s_setup):
        gbuf = (g0, g1, g2)
        pbuf = (p0, p1)
        sg = (sg0, sg1, sg2)
        sp = (sp0, sp1)
        so = (so0, so1, so2)

        wid = lax.axis_index("s") * NC + lax.axis_index("c")
        sq0 = wid * SEQW

        setup = []
        for b in range(B):
            setup.append(pltpu.make_async_copy(
                ids_hbm.at[b, pl.ds(sq0, SEQW)], idx2d.at[b], s_setup))
            setup.append(pltpu.make_async_copy(
                tt_hbm.at[b, pl.ds(sq0, SEQW)],
                tt2d.at[b, pl.ds(0, SEQW)], s_setup))
        setup.append(pltpu.make_async_copy(seg_hbm, seg_v, s_setup))
        for cp in setup:
            cp.start()

        def p_copy(c):
            return pltpu.make_async_copy(
                pe_hbm.at[pl.ds(sq0 + c * SEQCH, SEQCH)], pbuf[c % 2],
                sp[c % 2])

        p_copy(0).start()
        p_copy(1).start()

        for cp in setup:
            cp.wait()

        def g_copies(c):
            r = c % 3
            return [
                pltpu.make_async_copy(
                    table_hbm.at[idx2d.at[b, pl.ds(c * SEQCH, SEQCH)]],
                    gbuf[r].at[pl.ds(b * SEQCH, SEQCH)], sg[r])
                for b in range(B)
            ]

        def o_copies(c):
            r = c % 3
            return [
                pltpu.make_async_copy(
                    gbuf[r].at[pl.ds(b * SEQCH, SEQCH)],
                    out_hbm.at[b, pl.ds(sq0 + c * SEQCH, SEQCH)], so[r])
                for b in range(B)
            ]

        for cp in g_copies(0):
            cp.start()

        def dlt(j, _):
            sl = pl.ds(j * LANES, LANES)
            dlt_v[sl] = seg_v[1, sl] - seg_v[0, sl]
            return 0

        lax.fori_loop(0, DCH, dlt, 0)

        def bcast(ttv, k):
            return lax.gather(
                ttv, jnp.full((LANES, 1), k, jnp.int32), bcast_dnums, (1,),
                mode=lax.GatherScatterMode.PROMISE_IN_BOUNDS)

        def compute_batch(c, b):
            gb = gbuf[c % 3]
            pb = pbuf[c % 2]
            ttv = tt2d[b, pl.ds(c * SEQCH, LANES)].astype(jnp.float32)

            def dloop(j, _):
                sl = pl.ds(j * LANES, LANES)
                s0v = seg_v[0, sl]
                dv = dlt_v[sl]
                for k in range(SEQCH):
                    i = b * SEQCH + k
                    gb[i, sl] = (gb[i, sl] + (pb[k, sl] + s0v)
                                 + bcast(ttv, k) * dv)
                return 0

            lax.fori_loop(0, DCH, dloop, 0)

        def compute_all(c):
            gb = gbuf[c % 3]
            pb = pbuf[c % 2]
            ttvs = [tt2d[b, pl.ds(c * SEQCH, LANES)].astype(jnp.float32)
                    for b in range(B)]

            def dloop(j, _):
                sl = pl.ds(j * LANES, LANES)
                s0v = seg_v[0, sl]
                dv = dlt_v[sl]
                pek = [pb[k, sl] + s0v for k in range(SEQCH)]
                for b in range(B):
                    for k in range(SEQCH):
                        i = b * SEQCH + k
                        gb[i, sl] = (gb[i, sl] + pek[k]
                                     + bcast(ttvs[b], k) * dv)
                return 0

            lax.fori_loop(0, DCH, dloop, 0)

        for c in range(NCHUNK):
            if c >= 2:
                for cp in o_copies(c - 2):
                    cp.wait()
            if c + 1 < NCHUNK:
                for cp in g_copies(c + 1):
                    cp.start()
            head_tail = c == 0 or c == NCHUNK - 1
            if head_tail:
                p_copy(c).wait()
                gcs = g_copies(c)
                ocs = o_copies(c)
                for b in range(B):
                    gcs[b].wait()
                    compute_batch(c, b)
                    ocs[b].start()
            else:
                for cp in g_copies(c):
                    cp.wait()
                p_copy(c).wait()
                compute_all(c)
                for cp in o_copies(c):
                    cp.start()
            if c + 2 < NCHUNK:
                p_copy(c + 2).start()

        for c in (NCHUNK - 2, NCHUNK - 1):
            for cp in o_copies(c):
                cp.wait()

    return emb


def kernel(input_ids, token_type_ids, token_table, segment_table, pe):
    B, S = input_ids.shape
    V, D = token_table.shape
    TV = segment_table.shape[0]
    ids = input_ids if input_ids.dtype == jnp.int32 else (
        input_ids.astype(jnp.int32))
    tt = token_type_ids if token_type_ids.dtype == jnp.int32 else (
        token_type_ids.astype(jnp.int32))
    emb = _build(B, S, V, D, TV)
    return emb(ids, tt, token_table, segment_table, pe)


# speedup vs baseline: 2.0738x; 1.3240x over previous
import functools

import jax
import jax.numpy as jnp
from jax import lax
from jax.experimental import pallas as pl
from jax.experimental.pallas import tpu as pltpu
from jax.experimental.pallas import tpu_sc as plsc

LANES = 16


@functools.lru_cache(maxsize=None)
def _build(B, S, V, D, TV):
    info = plsc.get_sparse_core_info()
    NC, NS = info.num_cores, info.num_subcores
    NW = NC * NS
    assert S % NW == 0
    SEQW = S // NW
    SEQCH = 8
    assert SEQW % SEQCH == 0
    NCHUNK = SEQW // SEQCH
    DCH = D // LANES

    mesh = plsc.VectorSubcoreMesh(core_axis_name="c", subcore_axis_name="s")

    bcast_dnums = lax.GatherDimensionNumbers(
        offset_dims=(), collapsed_slice_dims=(0,), start_index_map=(0,))

    @functools.partial(
        pl.kernel,
        mesh=mesh,
        out_type=jax.ShapeDtypeStruct((B, S, D), jnp.float32),
        scratch_types=[
            pltpu.VMEM((B, SEQW), jnp.int32),
            pltpu.VMEM((B, SEQW + LANES), jnp.int32),
            pltpu.VMEM((B * SEQCH, D), jnp.float32),
            pltpu.VMEM((B * SEQCH, D), jnp.float32),
            pltpu.VMEM((B * SEQCH, D), jnp.float32),
            pltpu.VMEM((SEQCH, D), jnp.float32),
            pltpu.VMEM((SEQCH, D), jnp.float32),
            pltpu.VMEM((TV, D), jnp.float32),
            pltpu.VMEM((D,), jnp.float32),
            pltpu.SemaphoreType.DMA,
            pltpu.SemaphoreType.DMA,
            pltpu.SemaphoreType.DMA,
            pltpu.SemaphoreType.DMA,
            pltpu.SemaphoreType.DMA,
            pltpu.SemaphoreType.DMA,
            pltpu.SemaphoreType.DMA,
            pltpu.SemaphoreType.DMA,
            pltpu.SemaphoreType.DMA,
        ],
    )
    def emb(ids_hbm, tt_hbm, table_hbm, seg_hbm, pe_hbm, out_hbm,
            idx2d, tt2d, g0, g1, g2, p0, p1, seg_v, dlt_v,
            sg0, sg1, sg2, sp0, sp1, so0, so1, so2, s_setup):
        gbuf = (g0, g1, g2)
        pbuf = (p0, p1)
        sg = (sg0, sg1, sg2)
        sp = (sp0, sp1)
        so = (so0, so1, so2)

        wid = lax.axis_index("s") * NC + lax.axis_index("c")
        sq0 = wid * SEQW

        setup = []
        for b in range(B):
            setup.append(pltpu.make_async_copy(
                ids_hbm.at[b, pl.ds(sq0, SEQW)], idx2d.at[b], s_setup))
            setup.append(pltpu.make_async_copy(
                tt_hbm.at[b, pl.ds(sq0, SEQW)],
                tt2d.at[b, pl.ds(0, SEQW)], s_setup))
        setup.append(pltpu.make_async_copy(seg_hbm, seg_v, s_setup))
        for cp in setup:
            cp.start()

        def p_copy(c):
            return pltpu.make_async_copy(
                pe_hbm.at[pl.ds(sq0 + c * SEQCH, SEQCH)], pbuf[c % 2],
                sp[c % 2])

        p_copy(0).start()
        p_copy(1).start()

        for cp in setup:
            cp.wait()

        def g_copies(c):
            r = c % 3
            return [
                pltpu.make_async_copy(
                    table_hbm.at[idx2d.at[b, pl.ds(c * SEQCH, SEQCH)]],
                    gbuf[r].at[pl.ds(b * SEQCH, SEQCH)], sg[r])
                for b in range(B)
            ]

        def o_copies(c):
            r = c % 3
            return [
                pltpu.make_async_copy(
                    gbuf[r].at[pl.ds(b * SEQCH, SEQCH)],
                    out_hbm.at[b, pl.ds(sq0 + c * SEQCH, SEQCH)], so[r])
                for b in range(B)
            ]

        for cp in g_copies(0):
            cp.start()

        def dlt(j, _):
            sl = pl.ds(j * LANES, LANES)
            dlt_v[sl] = seg_v[1, sl] - seg_v[0, sl]
            return 0

        lax.fori_loop(0, DCH, dlt, 0)

        for c in range(NCHUNK):
            r = c % 3
            if c >= 2:
                for cp in o_copies(c - 2):
                    cp.wait()
            if c + 1 < NCHUNK:
                for cp in g_copies(c + 1):
                    cp.start()
            for cp in g_copies(c):
                cp.wait()
            p_copy(c).wait()

            gb = gbuf[r]
            pb = pbuf[c % 2]
            ttvs = [tt2d[b, pl.ds(c * SEQCH, LANES)].astype(jnp.float32)
                    for b in range(B)]

            def dloop(j, _):
                sl = pl.ds(j * LANES, LANES)
                s0v = seg_v[0, sl]
                dv = dlt_v[sl]
                pek = [pb[k, sl] + s0v for k in range(SEQCH)]
                for b in range(B):
                    for k in range(SEQCH):
                        f = lax.gather(
                            ttvs[b], jnp.full((LANES, 1), k, jnp.int32),
                            bcast_dnums, (1,),
                            mode=lax.GatherScatterMode.PROMISE_IN_BOUNDS)
                        i = b * SEQCH + k
                        gb[i, sl] = gb[i, sl] + pek[k] + f * dv
                return 0

            lax.fori_loop(0, DCH, dloop, 0)

            for cp in o_copies(c):
                cp.start()
            if c + 2 < NCHUNK:
                p_copy(c + 2).start()

        for c in (NCHUNK - 2, NCHUNK - 1):
            for cp in o_copies(c):
                cp.wait()

    return emb


def kernel(input_ids, token_type_ids, token_table, segment_table, pe):
    B, S = input_ids.shape
    V, D = token_table.shape
    TV = segment_table.shape[0]
    ids = input_ids if input_ids.dtype == jnp.int32 else (
        input_ids.astype(jnp.int32))
    tt = token_type_ids if token_type_ids.dtype == jnp.int32 else (
        token_type_ids.astype(jnp.int32))
    emb = _build(B, S, V, D, TV)
    return emb(ids, tt, token_table, segment_table, pe)


# speedup vs baseline: 2.1138x; 1.0193x over previous
import functools

import jax
import jax.numpy as jnp
from jax import lax
from jax.experimental import pallas as pl
from jax.experimental.pallas import tpu as pltpu
from jax.experimental.pallas import tpu_sc as plsc

LANES = 16


@functools.lru_cache(maxsize=None)
def _build(B, S, V, D, TV):
    info = plsc.get_sparse_core_info()
    NC, NS = info.num_cores, info.num_subcores
    NW = NC * NS
    assert S % NW == 0
    SEQW = S // NW
    SEQCH = 8
    assert SEQW % SEQCH == 0
    NCHUNK = SEQW // SEQCH
    DCH = D // LANES

    mesh = plsc.VectorSubcoreMesh(core_axis_name="c", subcore_axis_name="s")

    bcast_dnums = lax.GatherDimensionNumbers(
        offset_dims=(), collapsed_slice_dims=(0,), start_index_map=(0,))

    @functools.partial(
        pl.kernel,
        mesh=mesh,
        out_type=jax.ShapeDtypeStruct((B, S, D), jnp.float32),
        scratch_types=[
            pltpu.VMEM((B, SEQW), jnp.int32),
            pltpu.VMEM((B, SEQW + LANES), jnp.int32),
            pltpu.VMEM((B * SEQCH, D), jnp.float32),
            pltpu.VMEM((B * SEQCH, D), jnp.float32),
            pltpu.VMEM((B * SEQCH, D), jnp.float32),
            pltpu.VMEM((SEQCH, D), jnp.float32),
            pltpu.VMEM((SEQCH, D), jnp.float32),
            pltpu.VMEM((SEQCH, D), jnp.float32),
            pltpu.VMEM((TV, D), jnp.float32),
            pltpu.VMEM((D,), jnp.float32),
            pltpu.SemaphoreType.DMA,
            pltpu.SemaphoreType.DMA,
            pltpu.SemaphoreType.DMA,
            pltpu.SemaphoreType.DMA,
            pltpu.SemaphoreType.DMA,
            pltpu.SemaphoreType.DMA,
            pltpu.SemaphoreType.DMA,
            pltpu.SemaphoreType.DMA,
            pltpu.SemaphoreType.DMA,
            pltpu.SemaphoreType.DMA,
        ],
    )
    def emb(ids_hbm, tt_hbm, table_hbm, seg_hbm, pe_hbm, out_hbm,
            idx2d, tt2d, g0, g1, g2, p0, p1, p2, seg_v, dlt_v,
            sg0, sg1, sg2, sp0, sp1, sp2, so0, so1, so2, s_setup):
        gbuf = (g0, g1, g2)
        pbuf = (p0, p1, p2)
        sg = (sg0, sg1, sg2)
        sp = (sp0, sp1, sp2)
        so = (so0, so1, so2)

        wid = lax.axis_index("s") * NC + lax.axis_index("c")
        sq0 = wid * SEQW

        setup = []
        for b in range(B):
            setup.append(pltpu.make_async_copy(
                ids_hbm.at[b, pl.ds(sq0, SEQW)], idx2d.at[b], s_setup))
            setup.append(pltpu.make_async_copy(
                tt_hbm.at[b, pl.ds(sq0, SEQW)],
                tt2d.at[b, pl.ds(0, SEQW)], s_setup))
        setup.append(pltpu.make_async_copy(seg_hbm, seg_v, s_setup))
        for cp in setup:
            cp.start()

        def p_copy(c, rp):
            return pltpu.make_async_copy(
                pe_hbm.at[pl.ds(sq0 + c * SEQCH, SEQCH)], pbuf[rp], sp[rp])

        p_copy(0, 0).start()
        p_copy(1, 1).start()

        for cp in setup:
            cp.wait()

        def g_copies(c, r):
            return [
                pltpu.make_async_copy(
                    table_hbm.at[idx2d.at[b, pl.ds(c * SEQCH, SEQCH)]],
                    gbuf[r].at[pl.ds(b * SEQCH, SEQCH)], sg[r])
                for b in range(B)
            ]

        def o_copies(c, r):
            return [
                pltpu.make_async_copy(
                    gbuf[r].at[pl.ds(b * SEQCH, SEQCH)],
                    out_hbm.at[b, pl.ds(sq0 + c * SEQCH, SEQCH)], so[r])
                for b in range(B)
            ]

        for cp in g_copies(0, 0):
            cp.start()

        def dlt(j, _):
            sl = pl.ds(j * LANES, LANES)
            dlt_v[sl] = seg_v[1, sl] - seg_v[0, sl]
            return 0

        lax.fori_loop(0, DCH, dlt, 0)

        def compute_all(c, r, rp):
            gb = gbuf[r]
            pb = pbuf[rp]
            ttvs = [tt2d[b, pl.ds(c * SEQCH, LANES)].astype(jnp.float32)
                    for b in range(B)]

            def dloop(j, _):
                sl = pl.ds(j * LANES, LANES)
                s0v = seg_v[0, sl]
                dv = dlt_v[sl]
                pek = [pb[k, sl] + s0v for k in range(SEQCH)]
                for b in range(B):
                    for k in range(SEQCH):
                        f = lax.gather(
                            ttvs[b], jnp.full((LANES, 1), k, jnp.int32),
                            bcast_dnums, (1,),
                            mode=lax.GatherScatterMode.PROMISE_IN_BOUNDS)
                        i = b * SEQCH + k
                        gb[i, sl] = gb[i, sl] + pek[k] + f * dv
                return 0

            lax.fori_loop(0, DCH, dloop, 0)

        for cp in g_copies(1, 1):
            cp.start()
        for cp in g_copies(0, 0):
            cp.wait()
        p_copy(0, 0).wait()
        compute_all(0, 0, 0)
        for cp in o_copies(0, 0):
            cp.start()
        p_copy(2, 2).start()

        @pl.loop(1, NCHUNK - 1, step=3)
        def middle(base):
            for i in range(3):
                c = base + i
                r = (1 + i) % 3
                if i == 0:
                    @pl.when(c >= 2)
                    def _():
                        for cp in o_copies(c - 2, (r - 2) % 3):
                            cp.wait()
                else:
                    for cp in o_copies(c - 2, (r - 2) % 3):
                        cp.wait()
                for cp in g_copies(c + 1, (r + 1) % 3):
                    cp.start()
                for cp in g_copies(c, r):
                    cp.wait()
                p_copy(c, r).wait()
                compute_all(c, r, r)
                for cp in o_copies(c, r):
                    cp.start()
                if i == 2:
                    @pl.when(c + 2 < NCHUNK)
                    def _():
                        p_copy(c + 2, (r + 2) % 3).start()
                else:
                    p_copy(c + 2, (r + 2) % 3).start()

        for cp in o_copies(NCHUNK - 3, 2):
            cp.wait()
        for cp in g_copies(NCHUNK - 1, 1):
            cp.wait()
        p_copy(NCHUNK - 1, 1).wait()
        compute_all(NCHUNK - 1, 1, 1)
        for cp in o_copies(NCHUNK - 1, 1):
            cp.start()
        for cp in o_copies(NCHUNK - 2, 0):
            cp.wait()
        for cp in o_copies(NCHUNK - 1, 1):
            cp.wait()

    return emb


def kernel(input_ids, token_type_ids, token_table, segment_table, pe):
    B, S = input_ids.shape
    V, D = token_table.shape
    TV = segment_table.shape[0]
    ids = input_ids if input_ids.dtype == jnp.int32 else (
        input_ids.astype(jnp.int32))
    tt = token_type_ids if token_type_ids.dtype == jnp.int32 else (
        token_type_ids.astype(jnp.int32))
    emb = _build(B, S, V, D, TV)
    return emb(ids, tt, token_table, segment_table, pe)


# speedup vs baseline: 2.1542x; 1.0191x over previous
import functools

import jax
import jax.numpy as jnp
from jax import lax
from jax.experimental import pallas as pl
from jax.experimental.pallas import tpu as pltpu
from jax.experimental.pallas import tpu_sc as plsc

LANES = 16


@functools.lru_cache(maxsize=None)
def _build(B, S, V, D, TV):
    info = plsc.get_sparse_core_info()
    NC, NS = info.num_cores, info.num_subcores
    NW = NC * NS
    assert S % NW == 0
    SEQW = S // NW
    SEQCH = 8
    assert SEQW % SEQCH == 0
    NCHUNK = SEQW // SEQCH
    DCH = D // LANES

    mesh = plsc.VectorSubcoreMesh(core_axis_name="c", subcore_axis_name="s")

    bcast_dnums = lax.GatherDimensionNumbers(
        offset_dims=(), collapsed_slice_dims=(0,), start_index_map=(0,))

    @functools.partial(
        pl.kernel,
        mesh=mesh,
        out_type=jax.ShapeDtypeStruct((B, S, D), jnp.float32),
        scratch_types=[
            pltpu.VMEM((B, SEQW), jnp.int32),
            pltpu.VMEM((B, SEQW + LANES), jnp.int32),
            pltpu.VMEM((B * SEQCH, D), jnp.float32),
            pltpu.VMEM((B * SEQCH, D), jnp.float32),
            pltpu.VMEM((B * SEQCH, D), jnp.float32),
            pltpu.VMEM((SEQCH, D), jnp.float32),
            pltpu.VMEM((SEQCH, D), jnp.float32),
            pltpu.VMEM((SEQCH, D), jnp.float32),
            pltpu.VMEM((TV, D), jnp.float32),
            pltpu.VMEM((D,), jnp.float32),
            pltpu.SemaphoreType.DMA,
            pltpu.SemaphoreType.DMA,
            pltpu.SemaphoreType.DMA,
            pltpu.SemaphoreType.DMA,
            pltpu.SemaphoreType.DMA,
            pltpu.SemaphoreType.DMA,
            pltpu.SemaphoreType.DMA,
            pltpu.SemaphoreType.DMA,
            pltpu.SemaphoreType.DMA,
            pltpu.SemaphoreType.DMA,
        ],
    )
    def emb(ids_hbm, tt_hbm, table_hbm, seg_hbm, pe_hbm, out_hbm,
            idx2d, tt2d, g0, g1, g2, p0, p1, p2, seg_v, dlt_v,
            sg0, sg1, sg2, sp0, sp1, sp2, so0, so1, so2, s_setup):
        gbuf = (g0, g1, g2)
        pbuf = (p0, p1, p2)
        sg = (sg0, sg1, sg2)
        sp = (sp0, sp1, sp2)
        so = (so0, so1, so2)

        wid = lax.axis_index("s") * NC + lax.axis_index("c")
        sq0 = wid * SEQW

        setup_ids = [pltpu.make_async_copy(
            ids_hbm.at[b, pl.ds(sq0, SEQW)], idx2d.at[b], s_setup)
            for b in range(B)]
        setup_rest = [pltpu.make_async_copy(
            tt_hbm.at[b, pl.ds(sq0, SEQW)],
            tt2d.at[b, pl.ds(0, SEQW)], so[0]) for b in range(B)]
        setup_rest.append(pltpu.make_async_copy(seg_hbm, seg_v, so[0]))
        for cp in setup_ids:
            cp.start()
        for cp in setup_rest:
            cp.start()

        def p_copy(c, rp):
            return pltpu.make_async_copy(
                pe_hbm.at[pl.ds(sq0 + c * SEQCH, SEQCH)], pbuf[rp], sp[rp])

        p_copy(0, 0).start()
        p_copy(1, 1).start()

        def g_copies(c, r):
            return [
                pltpu.make_async_copy(
                    table_hbm.at[idx2d.at[b, pl.ds(c * SEQCH, SEQCH)]],
                    gbuf[r].at[pl.ds(b * SEQCH, SEQCH)], sg[r])
                for b in range(B)
            ]

        def o_copies(c, r):
            return [
                pltpu.make_async_copy(
                    gbuf[r].at[pl.ds(b * SEQCH, SEQCH)],
                    out_hbm.at[b, pl.ds(sq0 + c * SEQCH, SEQCH)], so[r])
                for b in range(B)
            ]

        for cp in setup_ids:
            cp.wait()
        for cp in g_copies(0, 0):
            cp.start()
        for cp in setup_rest:
            cp.wait()

        def dlt(j, _):
            sl = pl.ds(j * LANES, LANES)
            dlt_v[sl] = seg_v[1, sl] - seg_v[0, sl]
            return 0

        lax.fori_loop(0, DCH, dlt, 0)

        def compute_all(c, r, rp):
            gb = gbuf[r]
            pb = pbuf[rp]
            ttvs = [tt2d[b, pl.ds(c * SEQCH, LANES)].astype(jnp.float32)
                    for b in range(B)]

            def dloop(j, _):
                sl = pl.ds(j * LANES, LANES)
                s0v = seg_v[0, sl]
                dv = dlt_v[sl]
                pek = [pb[k, sl] + s0v for k in range(SEQCH)]
                for b in range(B):
                    for k in range(SEQCH):
                        f = lax.gather(
                            ttvs[b], jnp.full((LANES, 1), k, jnp.int32),
                            bcast_dnums, (1,),
                            mode=lax.GatherScatterMode.PROMISE_IN_BOUNDS)
                        i = b * SEQCH + k
                        gb[i, sl] = gb[i, sl] + pek[k] + f * dv
                return 0

            lax.fori_loop(0, DCH, dloop, 0)

        for cp in g_copies(1, 1):
            cp.start()
        for cp in g_copies(0, 0):
            cp.wait()
        p_copy(0, 0).wait()
        compute_all(0, 0, 0)
        for cp in o_copies(0, 0):
            cp.start()
        p_copy(2, 2).start()

        @pl.loop(1, NCHUNK - 1, step=3)
        def middle(base):
            for i in range(3):
                c = base + i
                r = (1 + i) % 3
                if i == 0:
                    @pl.when(c >= 2)
                    def _():
                        for cp in o_copies(c - 2, (r - 2) % 3):
                            cp.wait()
                else:
                    for cp in o_copies(c - 2, (r - 2) % 3):
                        cp.wait()
                for cp in g_copies(c + 1, (r + 1) % 3):
                    cp.start()
                for cp in g_copies(c, r):
                    cp.wait()
                p_copy(c, r).wait()
                compute_all(c, r, r)
                for cp in o_copies(c, r):
                    cp.start()
                if i == 2:
                    @pl.when(c + 2 < NCHUNK)
                    def _():
                        p_copy(c + 2, (r + 2) % 3).start()
                else:
                    p_copy(c + 2, (r + 2) % 3).start()

        for cp in o_copies(NCHUNK - 3, 2):
            cp.wait()
        for cp in g_copies(NCHUNK - 1, 1):
            cp.wait()
        p_copy(NCHUNK - 1, 1).wait()
        compute_all(NCHUNK - 1, 1, 1)
        for cp in o_copies(NCHUNK - 1, 1):
            cp.start()
        for cp in o_copies(NCHUNK - 2, 0):
            cp.wait()
        for cp in o_copies(NCHUNK - 1, 1):
            cp.wait()

    return emb


def kernel(input_ids, token_type_ids, token_table, segment_table, pe):
    B, S = input_ids.shape
    V, D = token_table.shape
    TV = segment_table.shape[0]
    ids = input_ids if input_ids.dtype == jnp.int32 else (
        input_ids.astype(jnp.int32))
    tt = token_type_ids if token_type_ids.dtype == jnp.int32 else (
        token_type_ids.astype(jnp.int32))
    emb = _build(B, S, V, D, TV)
    return emb(ids, tt, token_table, segment_table, pe)
